# baseline (speedup 1.0000x reference)
import functools

import jax
import jax.numpy as jnp
from jax import lax
from jax.experimental import pallas as pl
from jax.experimental.pallas import tpu as pltpu
from jax.experimental.pallas import tpu_sc as plsc

NC = 2
NS = 16
L = 16
W_E = 128
WIN_T = 80
N_PAD = 10240
CH = 16
SL = 2 * CH
TPW = WIN_T * W_E


def _mm_body(x_ref, w_ref, o_ref):
    o_ref[...] = lax.dot_general(
        x_ref[...], w_ref[...], (((1,), (1,)), ((), ())),
        preferred_element_type=jnp.float32)


def _matmul(x, w):
    n, d = x.shape
    blk = 1000
    return pl.pallas_call(
        _mm_body,
        grid=(n // blk,),
        in_specs=[pl.BlockSpec((blk, d), lambda i: (i, 0)),
                  pl.BlockSpec((d, d), lambda i: (0, 0))],
        out_specs=pl.BlockSpec((blk, d), lambda i: (i, 0)),
        out_shape=jax.ShapeDtypeStruct((n, d), jnp.float32),
    )(x, w)


def _fin_body(x_ref, h_ref, a0_ref, a1_ref, dis_ref, b_ref, o_ref):
    dv = dis_ref[...]
    s = a0_ref[...] + a1_ref[...] + dv * h_ref[...]
    o_ref[...] = jnp.maximum(dv * s + b_ref[...], 0.0) + x_ref[...]


def _finalize(x, h, a0, a1, dis2d, b2d):
    n, d = x.shape
    blk = 1000
    rows = pl.BlockSpec((blk, d), lambda i: (i, 0))
    return pl.pallas_call(
        _fin_body,
        grid=(n // blk,),
        in_specs=[rows, rows, rows, rows,
                  pl.BlockSpec((blk, 1), lambda i: (i, 0)),
                  pl.BlockSpec((1, d), lambda i: (0, 0))],
        out_specs=rows,
        out_shape=jax.ShapeDtypeStruct((n, d), jnp.float32),
    )(x, h, a0, a1, dis2d, b2d)


def _rsqrt16(d):
    i = lax.bitcast_convert_type(d, jnp.int32)
    r = lax.bitcast_convert_type(jnp.int32(0x5F3759DF) - (i >> 1), jnp.float32)
    for _ in range(3):
        r = r * (1.5 - 0.5 * d * r * r)
    return r


def _make_sc(n, d):
    deg_rounds = 2
    rows_t = N_PAD // NS
    nchk = WIN_T // CH
    ce = CH * W_E
    mesh = plsc.VectorSubcoreMesh(
        core_axis_name="c", subcore_axis_name="s",
        num_cores=NC, num_subcores=NS)

    @functools.partial(
        pl.kernel,
        out_type=(jax.ShapeDtypeStruct((NC, N_PAD, d), jnp.float32),
                  jax.ShapeDtypeStruct((N_PAD,), jnp.float32)),
        mesh=mesh,
        scratch_types=dict(
            row_v=pltpu.VMEM((SL * W_E,), jnp.int32),
            col_v=pltpu.VMEM((SL * W_E,), jnp.int32),
            ew_v=pltpu.VMEM((SL * W_E,), jnp.float32),
            hb0=pltpu.VMEM((W_E, d), jnp.float32),
            hb1=pltpu.VMEM((W_E, d), jnp.float32),
            dbuf=pltpu.VMEM((W_E,), jnp.float32),
            zd=pltpu.VMEM((N_PAD // NS,), jnp.float32),
            agg_sh=pltpu.VMEM_SHARED((N_PAD, d), jnp.float32),
            deg_sh=pltpu.VMEM_SHARED((N_PAD,), jnp.float32),
            gsem=pltpu.SemaphoreType.DMA,
            esem=pltpu.SemaphoreType.DMA,
            ssem=pltpu.SemaphoreType.DMA,
        ),
    )
    def sc(row1, col1, ew1, h_hbm, aggp, dis_out, *, row_v,
           col_v, ew_v, hb0, hb1, dbuf, zd, agg_sh, deg_sh, gsem, esem, ssem):
        c = lax.axis_index("c")
        s = lax.axis_index("s")
        zv = jnp.zeros((L,), jnp.float32)

        def _zb(i, _):
            for q in range(d // L):
                hb0[i, pl.ds(q * L, L)] = zv
            return 0
        lax.fori_loop(0, W_E, _zb, 0)

        def _zd(i, _):
            zd[pl.ds(i * L, L)] = zv
            return 0
        lax.fori_loop(0, (N_PAD // NS) // L, _zd, 0)
        for k in range(rows_t // W_E):
            pltpu.sync_copy(hb0, agg_sh.at[pl.ds(s * rows_t + k * W_E, W_E)])
        pltpu.sync_copy(zd, deg_sh.at[pl.ds(s * (N_PAD // NS), N_PAD // NS)])
        plsc.subcore_barrier()

        ebase = (c * NS + s) * TPW

        nchka = deg_rounds * (WIN_T // CH)

        def _a_src(j):
            rnd, kk = j // (WIN_T // CH), j % (WIN_T // CH)
            return (deg_rounds * s + rnd) * TPW + kk * ce

        def _a_load(j):
            src = _a_src(j)
            dst = pl.ds((j % 2) * ce, ce)
            pltpu.async_copy(col1.at[pl.ds(src, ce)], col_v.at[dst], esem)
            pltpu.async_copy(ew1.at[pl.ds(src, ce)], ew_v.at[dst], esem)

        def _a_load_wait(j):
            src = _a_src(j)
            dst = pl.ds((j % 2) * ce, ce)
            pltpu.make_async_copy(col1.at[pl.ds(src, ce)],
                                  col_v.at[dst], esem).wait()
            pltpu.make_async_copy(ew1.at[pl.ds(src, ce)],
                                  ew_v.at[dst], esem).wait()

        def _a_scat_ref(j):
            half = pl.ds((j % 2) * ce, ce)
            return pltpu.make_async_copy(
                ew_v.at[half], deg_sh.at[col_v.at[half]], ssem)

        _a_load(0)
        for j in range(nchka):
            _a_load_wait(j)
            pltpu.async_copy(ew_v.at[pl.ds((j % 2) * ce, ce)],
                             deg_sh.at[col_v.at[pl.ds((j % 2) * ce, ce)]],
                             ssem, add=True)
            if j >= 1:
                _a_scat_ref(j - 1).wait()
            if j + 1 < nchka:
                _a_load(j + 1)
        _a_scat_ref(nchka - 1).wait()
        pltpu.async_copy(row1.at[pl.ds(ebase, ce)],
                         row_v.at[pl.ds(0, ce)], esem)
        pltpu.async_copy(col1.at[pl.ds(ebase, ce)],
                         col_v.at[pl.ds(0, ce)], esem)
        pltpu.async_copy(ew1.at[pl.ds(ebase, ce)],
                         ew_v.at[pl.ds(0, ce)], esem)
        plsc.subcore_barrier()

        chunk = N_PAD // NS
        pltpu.sync_copy(deg_sh.at[pl.ds(s * chunk, chunk)], zd)

        def _dis(i, _):
            dv = zd[pl.ds(i * L, L)] + 1.0
            zd[pl.ds(i * L, L)] = _rsqrt16(dv)
            return 0
        lax.fori_loop(0, chunk // L, _dis, 0)
        pltpu.sync_copy(zd, deg_sh.at[pl.ds(s * chunk, chunk)])
        plsc.subcore_barrier()

        def _refill_issue(k):
            src = pl.ds(ebase + k * ce, ce)
            dst = pl.ds((k % 2) * ce, ce)
            pltpu.async_copy(row1.at[src], row_v.at[dst], esem)
            pltpu.async_copy(col1.at[src], col_v.at[dst], esem)
            pltpu.async_copy(ew1.at[src], ew_v.at[dst], esem)

        def _refill_wait(k):
            src = pl.ds(ebase + k * ce, ce)
            dst = pl.ds((k % 2) * ce, ce)
            pltpu.make_async_copy(row1.at[src], row_v.at[dst], esem).wait()
            pltpu.make_async_copy(col1.at[src], col_v.at[dst], esem).wait()
            pltpu.make_async_copy(ew1.at[src], ew_v.at[dst], esem).wait()

        def _gissue(slot, buf):
            idx = row_v.at[pl.ds(slot * W_E, W_E)]
            pltpu.async_copy(h_hbm.at[idx], buf, gsem)

        def _scat_ref(slot, buf):
            return pltpu.make_async_copy(
                buf, agg_sh.at[col_v.at[pl.ds(slot * W_E, W_E)]], ssem)

        def _process(slot, cur, nxt_slot, prev_slot):
            nxt = hb1 if cur is hb0 else hb0
            idx = row_v.at[pl.ds(slot * W_E, W_E)]
            pltpu.make_async_copy(h_hbm.at[idx], cur, gsem).wait()
            if prev_slot is not None:
                _scat_ref(prev_slot, nxt).wait()
            if nxt_slot is not None:
                _gissue(nxt_slot, nxt)
            pltpu.sync_copy(deg_sh.at[row_v.at[pl.ds(slot * W_E, W_E)]], dbuf)

            def _grp(g, _):
                s16 = (ew_v[pl.ds(slot * W_E + g * L, L)]
                       * dbuf[pl.ds(g * L, L)])
                for r in range(L):
                    sp = lax.gather(
                        s16, jnp.full((L, 1), r, jnp.int32),
                        dimension_numbers=lax.GatherDimensionNumbers(
                            offset_dims=(), collapsed_slice_dims=(0,),
                            start_index_map=(0,)),
                        slice_sizes=(1,),
                        mode=lax.GatherScatterMode.PROMISE_IN_BOUNDS)
                    rr = g * L + r
                    for q in range(d // L):
                        cur[rr, pl.ds(q * L, L)] = cur[rr, pl.ds(q * L, L)] * sp
                return 0
            lax.fori_loop(0, W_E // L, _grp, 0)
            pltpu.async_copy(cur,
                             agg_sh.at[col_v.at[pl.ds(slot * W_E, W_E)]],
                             ssem, add=True)

        pltpu.make_async_copy(row1.at[pl.ds(ebase, ce)],
                              row_v.at[pl.ds(0, ce)], esem).wait()
        pltpu.make_async_copy(col1.at[pl.ds(ebase, ce)],
                              col_v.at[pl.ds(0, ce)], esem).wait()
        pltpu.make_async_copy(ew1.at[pl.ds(ebase, ce)],
                              ew_v.at[pl.ds(0, ce)], esem).wait()
        _gissue(0, hb0)

        for k in range(nchk):
            base = (k % 2) * CH
            pb = ((k + 1) % 2) * CH
            last = k == nchk - 1

            _process(base, hb0, base + 1, None if k == 0 else pb + CH - 1)
            if not last:
                _refill_issue(k + 1)
            _process(base + 1, hb1, base + 2, base)

            def _pair(p, _):
                i0 = base + 2 + 2 * p
                _process(i0, hb0, i0 + 1, i0 - 1)
                _process(i0 + 1, hb1, i0 + 2, i0)
                return 0
            lax.fori_loop(0, CH // 2 - 2, _pair, 0)

            _process(base + CH - 2, hb0, base + CH - 1, base + CH - 3)
            if not last:
                _refill_wait(k + 1)
                _process(base + CH - 1, hb1, pb, base + CH - 2)
            else:
                _process(base + CH - 1, hb1, None, base + CH - 2)
        _scat_ref(((nchk - 1) % 2) * CH + CH - 1, hb1).wait()
        plsc.subcore_barrier()

        for k in range(rows_t // W_E):
            off = s * rows_t + k * W_E
            pltpu.async_copy(agg_sh.at[pl.ds(off, W_E)],
                             aggp.at[c, pl.ds(off, W_E)], gsem)
        for k in range(rows_t // W_E):
            off = s * rows_t + k * W_E
            pltpu.make_async_copy(agg_sh.at[pl.ds(off, W_E)],
                                  aggp.at[c, pl.ds(off, W_E)], gsem).wait()

        @pl.when(c == 0)
        def _():
            pltpu.sync_copy(deg_sh.at[pl.ds(s * chunk, chunk)],
                            dis_out.at[pl.ds(s * chunk, chunk)])

    return sc


def kernel(x, edge_index, edge_weight, W, b):
    n, d = x.shape
    e = edge_weight.shape[0]
    nw = NC * NS
    e_pad = nw * WIN_T * W_E
    pad = e_pad - e

    pad_idx = jnp.arange(pad, dtype=edge_index.dtype) % n
    row_p = jnp.concatenate([edge_index[0], pad_idx])
    col_p = jnp.concatenate([edge_index[1], pad_idx])
    ew_p = jnp.concatenate(
        [edge_weight, jnp.zeros((pad,), edge_weight.dtype)])

    h = _matmul(x, W)
    aggp, dis_pad = _make_sc(n, d)(row_p, col_p, ew_p, h)
    dis2d = dis_pad[:n].reshape(n, 1)
    return _finalize(x, h, aggp[0, :n], aggp[1, :n], dis2d, b.reshape(1, d))


# speedup vs baseline: 1.0296x; 1.0296x over previous
import functools

import jax
import jax.numpy as jnp
from jax import lax
from jax.experimental import pallas as pl
from jax.experimental.pallas import tpu as pltpu
from jax.experimental.pallas import tpu_sc as plsc

NC = 2
NS = 16
L = 16
W_E = 128
WIN_T = 80
N_PAD = 10240
CH = 16
SL = 2 * CH
TPW = WIN_T * W_E


def _mm_body(x_ref, w_ref, o_ref):
    o_ref[...] = lax.dot_general(
        x_ref[...], w_ref[...], (((1,), (1,)), ((), ())),
        preferred_element_type=jnp.float32)


def _matmul(x, w):
    n, d = x.shape
    blk = 1000
    return pl.pallas_call(
        _mm_body,
        grid=(n // blk,),
        in_specs=[pl.BlockSpec((blk, d), lambda i: (i, 0)),
                  pl.BlockSpec((d, d), lambda i: (0, 0))],
        out_specs=pl.BlockSpec((blk, d), lambda i: (i, 0)),
        out_shape=jax.ShapeDtypeStruct((n, d), jnp.float32),
    )(x, w)


def _fin_body(x_ref, h_ref, a0_ref, a1_ref, dis_ref, b_ref, o_ref):
    dv = dis_ref[...]
    s = a0_ref[...] + a1_ref[...] + dv * h_ref[...]
    o_ref[...] = jnp.maximum(dv * s + b_ref[...], 0.0) + x_ref[...]


def _finalize(x, h, a0, a1, dis2d, b2d):
    n, d = x.shape
    blk = 1000
    rows = pl.BlockSpec((blk, d), lambda i: (i, 0))
    return pl.pallas_call(
        _fin_body,
        grid=(n // blk,),
        in_specs=[rows, rows, rows, rows,
                  pl.BlockSpec((blk, 1), lambda i: (i, 0)),
                  pl.BlockSpec((1, d), lambda i: (0, 0))],
        out_specs=rows,
        out_shape=jax.ShapeDtypeStruct((n, d), jnp.float32),
    )(x, h, a0, a1, dis2d, b2d)


def _rsqrt16(d):
    i = lax.bitcast_convert_type(d, jnp.int32)
    r = lax.bitcast_convert_type(jnp.int32(0x5F3759DF) - (i >> 1), jnp.float32)
    for _ in range(3):
        r = r * (1.5 - 0.5 * d * r * r)
    return r


def _make_sc(n, d):
    deg_rounds = 2
    rows_t = N_PAD // NS
    nchk = WIN_T // CH
    ce = CH * W_E
    mesh = plsc.VectorSubcoreMesh(
        core_axis_name="c", subcore_axis_name="s",
        num_cores=NC, num_subcores=NS)

    @functools.partial(
        pl.kernel,
        out_type=(jax.ShapeDtypeStruct((NC, N_PAD, d), jnp.float32),
                  jax.ShapeDtypeStruct((N_PAD,), jnp.float32)),
        mesh=mesh,
        scratch_types=dict(
            row_v=pltpu.VMEM((SL * W_E,), jnp.int32),
            col_v=pltpu.VMEM((SL * W_E,), jnp.int32),
            ew_v=pltpu.VMEM((SL * W_E,), jnp.float32),
            hb0=pltpu.VMEM((W_E, d), jnp.float32),
            hb1=pltpu.VMEM((W_E, d), jnp.float32),
            dbuf=pltpu.VMEM((W_E,), jnp.float32),
            dbuf1=pltpu.VMEM((W_E,), jnp.float32),
            zd=pltpu.VMEM((N_PAD // NS,), jnp.float32),
            agg_sh=pltpu.VMEM_SHARED((N_PAD, d), jnp.float32),
            deg_sh=pltpu.VMEM_SHARED((N_PAD,), jnp.float32),
            gsem=pltpu.SemaphoreType.DMA,
            esem=pltpu.SemaphoreType.DMA,
            ssem=pltpu.SemaphoreType.DMA,
            dsem=pltpu.SemaphoreType.DMA,
        ),
    )
    def sc(row1, col1, ew1, h_hbm, aggp, dis_out, *, row_v,
           col_v, ew_v, hb0, hb1, dbuf, dbuf1, zd, agg_sh, deg_sh,
           gsem, esem, ssem, dsem):
        c = lax.axis_index("c")
        s = lax.axis_index("s")
        zv = jnp.zeros((L,), jnp.float32)

        def _zb(i, _):
            for q in range(d // L):
                hb0[i, pl.ds(q * L, L)] = zv
            return 0
        lax.fori_loop(0, W_E, _zb, 0)

        def _zd(i, _):
            zd[pl.ds(i * L, L)] = zv
            return 0
        lax.fori_loop(0, (N_PAD // NS) // L, _zd, 0)
        for k in range(rows_t // W_E):
            pltpu.sync_copy(hb0, agg_sh.at[pl.ds(s * rows_t + k * W_E, W_E)])
        pltpu.sync_copy(zd, deg_sh.at[pl.ds(s * (N_PAD // NS), N_PAD // NS)])
        plsc.subcore_barrier()

        ebase = (c * NS + s) * TPW

        nchka = deg_rounds * (WIN_T // CH)

        def _a_src(j):
            rnd, kk = j // (WIN_T // CH), j % (WIN_T // CH)
            return (deg_rounds * s + rnd) * TPW + kk * ce

        def _a_load(j):
            src = _a_src(j)
            dst = pl.ds((j % 2) * ce, ce)
            pltpu.async_copy(col1.at[pl.ds(src, ce)], col_v.at[dst], esem)
            pltpu.async_copy(ew1.at[pl.ds(src, ce)], ew_v.at[dst], esem)

        def _a_load_wait(j):
            src = _a_src(j)
            dst = pl.ds((j % 2) * ce, ce)
            pltpu.make_async_copy(col1.at[pl.ds(src, ce)],
                                  col_v.at[dst], esem).wait()
            pltpu.make_async_copy(ew1.at[pl.ds(src, ce)],
                                  ew_v.at[dst], esem).wait()

        def _a_scat_ref(j):
            half = pl.ds((j % 2) * ce, ce)
            return pltpu.make_async_copy(
                ew_v.at[half], deg_sh.at[col_v.at[half]], ssem)

        _a_load(0)
        for j in range(nchka):
            _a_load_wait(j)
            pltpu.async_copy(ew_v.at[pl.ds((j % 2) * ce, ce)],
                             deg_sh.at[col_v.at[pl.ds((j % 2) * ce, ce)]],
                             ssem, add=True)
            if j >= 1:
                _a_scat_ref(j - 1).wait()
            if j + 1 < nchka:
                _a_load(j + 1)
        _a_scat_ref(nchka - 1).wait()
        pltpu.async_copy(row1.at[pl.ds(ebase, ce)],
                         row_v.at[pl.ds(0, ce)], esem)
        pltpu.async_copy(col1.at[pl.ds(ebase, ce)],
                         col_v.at[pl.ds(0, ce)], esem)
        pltpu.async_copy(ew1.at[pl.ds(ebase, ce)],
                         ew_v.at[pl.ds(0, ce)], esem)
        plsc.subcore_barrier()

        chunk = N_PAD // NS
        pltpu.sync_copy(deg_sh.at[pl.ds(s * chunk, chunk)], zd)

        def _dis(i, _):
            dv = zd[pl.ds(i * L, L)] + 1.0
            zd[pl.ds(i * L, L)] = _rsqrt16(dv)
            return 0
        lax.fori_loop(0, chunk // L, _dis, 0)
        pltpu.sync_copy(zd, deg_sh.at[pl.ds(s * chunk, chunk)])
        plsc.subcore_barrier()

        def _refill_issue(k):
            src = pl.ds(ebase + k * ce, ce)
            dst = pl.ds((k % 2) * ce, ce)
            pltpu.async_copy(row1.at[src], row_v.at[dst], esem)
            pltpu.async_copy(col1.at[src], col_v.at[dst], esem)
            pltpu.async_copy(ew1.at[src], ew_v.at[dst], esem)

        def _refill_wait(k):
            src = pl.ds(ebase + k * ce, ce)
            dst = pl.ds((k % 2) * ce, ce)
            pltpu.make_async_copy(row1.at[src], row_v.at[dst], esem).wait()
            pltpu.make_async_copy(col1.at[src], col_v.at[dst], esem).wait()
            pltpu.make_async_copy(ew1.at[src], ew_v.at[dst], esem).wait()

        def _gissue(slot, buf):
            idx = row_v.at[pl.ds(slot * W_E, W_E)]
            pltpu.async_copy(h_hbm.at[idx], buf, gsem)

        def _scat_ref(slot, buf):
            return pltpu.make_async_copy(
                buf, agg_sh.at[col_v.at[pl.ds(slot * W_E, W_E)]], ssem)

        def _dissue(slot, db):
            idx = row_v.at[pl.ds(slot * W_E, W_E)]
            pltpu.async_copy(deg_sh.at[idx], db, dsem)

        def _process(slot, cur, nxt_slot, prev_slot):
            nxt = hb1 if cur is hb0 else hb0
            dbc = dbuf if cur is hb0 else dbuf1
            dbn = dbuf1 if cur is hb0 else dbuf
            idx = row_v.at[pl.ds(slot * W_E, W_E)]
            pltpu.make_async_copy(h_hbm.at[idx], cur, gsem).wait()
            if prev_slot is not None:
                _scat_ref(prev_slot, nxt).wait()
            if nxt_slot is not None:
                _gissue(nxt_slot, nxt)
                _dissue(nxt_slot, dbn)
            pltpu.make_async_copy(deg_sh.at[idx], dbc, dsem).wait()

            def _grp(g, _):
                s16 = (ew_v[pl.ds(slot * W_E + g * L, L)]
                       * dbc[pl.ds(g * L, L)])
                for r in range(L):
                    sp = lax.gather(
                        s16, jnp.full((L, 1), r, jnp.int32),
                        dimension_numbers=lax.GatherDimensionNumbers(
                            offset_dims=(), collapsed_slice_dims=(0,),
                            start_index_map=(0,)),
                        slice_sizes=(1,),
                        mode=lax.GatherScatterMode.PROMISE_IN_BOUNDS)
                    rr = g * L + r
                    for q in range(d // L):
                        cur[rr, pl.ds(q * L, L)] = cur[rr, pl.ds(q * L, L)] * sp
                return 0
            lax.fori_loop(0, W_E // L, _grp, 0)
            pltpu.async_copy(cur,
                             agg_sh.at[col_v.at[pl.ds(slot * W_E, W_E)]],
                             ssem, add=True)

        pltpu.make_async_copy(row1.at[pl.ds(ebase, ce)],
                              row_v.at[pl.ds(0, ce)], esem).wait()
        pltpu.make_async_copy(col1.at[pl.ds(ebase, ce)],
                              col_v.at[pl.ds(0, ce)], esem).wait()
        pltpu.make_async_copy(ew1.at[pl.ds(ebase, ce)],
                              ew_v.at[pl.ds(0, ce)], esem).wait()
        _gissue(0, hb0)
        _dissue(0, dbuf)

        for k in range(nchk):
            base = (k % 2) * CH
            pb = ((k + 1) % 2) * CH
            last = k == nchk - 1

            _process(base, hb0, base + 1, None if k == 0 else pb + CH - 1)
            if not last:
                _refill_issue(k + 1)
            _process(base + 1, hb1, base + 2, base)

            def _pair(p, _):
                i0 = base + 2 + 2 * p
                _process(i0, hb0, i0 + 1, i0 - 1)
                _process(i0 + 1, hb1, i0 + 2, i0)
                return 0
            lax.fori_loop(0, CH // 2 - 2, _pair, 0)

            _process(base + CH - 2, hb0, base + CH - 1, base + CH - 3)
            if not last:
                _refill_wait(k + 1)
                _process(base + CH - 1, hb1, pb, base + CH - 2)
            else:
                _process(base + CH - 1, hb1, None, base + CH - 2)
        _scat_ref(((nchk - 1) % 2) * CH + CH - 1, hb1).wait()
        plsc.subcore_barrier()

        for k in range(rows_t // W_E):
            off = s * rows_t + k * W_E
            pltpu.async_copy(agg_sh.at[pl.ds(off, W_E)],
                             aggp.at[c, pl.ds(off, W_E)], gsem)
        for k in range(rows_t // W_E):
            off = s * rows_t + k * W_E
            pltpu.make_async_copy(agg_sh.at[pl.ds(off, W_E)],
                                  aggp.at[c, pl.ds(off, W_E)], gsem).wait()

        @pl.when(c == 0)
        def _():
            pltpu.sync_copy(deg_sh.at[pl.ds(s * chunk, chunk)],
                            dis_out.at[pl.ds(s * chunk, chunk)])

    return sc


def kernel(x, edge_index, edge_weight, W, b):
    n, d = x.shape
    e = edge_weight.shape[0]
    nw = NC * NS
    e_pad = nw * WIN_T * W_E
    pad = e_pad - e

    pad_idx = jnp.arange(pad, dtype=edge_index.dtype) % n
    row_p = jnp.concatenate([edge_index[0], pad_idx])
    col_p = jnp.concatenate([edge_index[1], pad_idx])
    ew_p = jnp.concatenate(
        [edge_weight, jnp.zeros((pad,), edge_weight.dtype)])

    h = _matmul(x, W)
    aggp, dis_pad = _make_sc(n, d)(row_p, col_p, ew_p, h)
    dis2d = dis_pad[:n].reshape(n, 1)
    return _finalize(x, h, aggp[0, :n], aggp[1, :n], dis2d, b.reshape(1, d))
